# baseline (speedup 1.0000x reference)
import functools

import jax
import jax.numpy as jnp
from jax import lax
from jax.experimental import pallas as pl
from jax.experimental.pallas import tpu as pltpu
from jax.experimental.pallas import tpu_sc as plsc

T = 2048
D = 2048
F = 1024
E = 8
K = 2
TM = 256
NT = (K * T) // TM + (E - 1)
G = NT * TM
NW = 32
CH = 8

_HIGH = jax.lax.Precision.HIGHEST


def _router_body(x_ref, gw_ref, bias_ref, pos_ref, wn_ref, teid_ref,
                 src_ref, wsf_ref):
  x = x_ref[...]
  gw = gw_ref[...]
  logits = lax.dot_general(gw, x, (((1,), (1,)), ((), ())),
                           preferred_element_type=jnp.float32)
  s = jax.nn.sigmoid(logits)
  scores = s + bias_ref[:, 0:1]

  eidx = lax.broadcasted_iota(jnp.int32, (E, T), 0)
  m1 = jnp.max(scores, axis=0, keepdims=True)
  i1 = jnp.min(jnp.where(scores == m1, eidx, E), axis=0, keepdims=True)
  sc2 = jnp.where(eidx == i1, -jnp.inf, scores)
  m2 = jnp.max(sc2, axis=0, keepdims=True)
  i2 = jnp.min(jnp.where(sc2 == m2, eidx, E), axis=0, keepdims=True)

  w1r = jnp.sum(jnp.where(eidx == i1, s, 0.0), axis=0, keepdims=True)
  w2r = jnp.sum(jnp.where(eidx == i2, s, 0.0), axis=0, keepdims=True)
  wsum = w1r + w2r
  wn0 = w1r / wsum
  wn1 = w2r / wsum

  ids = jnp.concatenate([i1, i2], axis=0)
  masks = (ids[None, :, :] ==
           lax.broadcasted_iota(jnp.int32, (E, K, T), 0))
  masks16 = masks.reshape(E * K, T).astype(jnp.float32)

  r_iota = lax.broadcasted_iota(jnp.int32, (T, T), 0)
  c_iota = lax.broadcasted_iota(jnp.int32, (T, T), 1)
  tri = (r_iota >= c_iota).astype(jnp.float32)
  cums16 = lax.dot_general(masks16, tri, (((1,), (1,)), ((), ())),
                           preferred_element_type=jnp.float32,
                           precision=_HIGH)

  g256 = lax.broadcasted_iota(jnp.int32, (1, 128), 1).astype(
      jnp.float32) * float(TM)
  pos0 = jnp.zeros((1, T), jnp.float32)
  pos1 = jnp.zeros((1, T), jnp.float32)
  teid_f = jnp.zeros((1, 128), jnp.float32)
  off = jnp.zeros((1, 1), jnp.float32)
  for e in range(E):
    m0 = masks16[2 * e:2 * e + 1, :]
    m1e = masks16[2 * e + 1:2 * e + 2, :]
    cu0 = cums16[2 * e:2 * e + 1, :]
    cu1 = cums16[2 * e + 1:2 * e + 2, :]
    c0 = cu0[:, T - 1:T]
    c1 = cu1[:, T - 1:T]
    pad = jnp.ceil((c0 + c1) / float(TM)) * float(TM)
    pos0 = pos0 + m0 * (off + cu0 - 1.0)
    pos1 = pos1 + m1e * (off + c0 + cu1 - 1.0)
    end = off + pad
    teid_f = teid_f + (g256 >= end).astype(jnp.float32)
    off = end
  teid_ref[...] = jnp.minimum(teid_f, float(E - 1)).astype(jnp.int32)
  pos_ref[0:1, :] = pos0.astype(jnp.int32)
  pos_ref[1:2, :] = pos1.astype(jnp.int32)
  wn_ref[0:1, :] = wn0
  wn_ref[1:2, :] = wn1

  t_iota = lax.broadcasted_iota(jnp.int32, (1, T), 1).astype(jnp.float32)
  pos_flat = jnp.concatenate([pos0, pos1], axis=1)
  tvec = jnp.concatenate([t_iota, t_iota], axis=1)
  wvec = jnp.concatenate([wn0, wn1], axis=1)
  rhs = jnp.concatenate([tvec, wvec], axis=0)
  p_col = lax.broadcasted_iota(jnp.int32, (TM, 1), 0).astype(jnp.float32)
  for g in range(NT):
    eq = (p_col + float(g * TM) == pos_flat).astype(jnp.float32)
    sw = lax.dot_general(eq, rhs, (((1,), (1,)), ((), ())),
                         preferred_element_type=jnp.float32,
                         precision=_HIGH)
    src_ref[g * TM:(g + 1) * TM, :] = sw[:, 0:1].astype(jnp.int32)
    wsf_ref[g * TM:(g + 1) * TM, :] = sw[:, 1:2]


def _router_call(x, gate_w, bias_b):
  return pl.pallas_call(
      _router_body,
      out_shape=(
          jax.ShapeDtypeStruct((K, T), jnp.int32),
          jax.ShapeDtypeStruct((K, T), jnp.float32),
          jax.ShapeDtypeStruct((1, 128), jnp.int32),
          jax.ShapeDtypeStruct((G, 1), jnp.int32),
          jax.ShapeDtypeStruct((G, 1), jnp.float32),
      ),
  )(x, gate_w, bias_b)


@functools.lru_cache(maxsize=None)
def _make_sc_gather(n_chunks):
  n_rows = NW * n_chunks * CH
  mesh = plsc.VectorSubcoreMesh(core_axis_name="c", subcore_axis_name="s",
                                num_cores=2, num_subcores=16)

  def body(table_hbm, idx_hbm, out_hbm, idx_v, buf0, buf1, g0, g1, o0, o1):
    wid = lax.axis_index("s") * 2 + lax.axis_index("c")
    base = wid * (n_chunks * CH)
    pltpu.sync_copy(idx_hbm.at[wid], idx_v)
    bufs = (buf0, buf1)
    gsem = (g0, g1)
    osem = (o0, o1)
    cps = [None, None]
    sps = [None, None]
    for i in range(n_chunks):
      b = i & 1
      if sps[b] is not None:
        sps[b].wait()
      cps[b] = pltpu.async_copy(table_hbm.at[idx_v.at[i]], bufs[b], gsem[b])
      if i > 0:
        pb = (i - 1) & 1
        cps[pb].wait()
        sps[pb] = pltpu.async_copy(
            bufs[pb], out_hbm.at[pl.ds(base + (i - 1) * CH, CH)], osem[pb])
    lb = (n_chunks - 1) & 1
    cps[lb].wait()
    sps[lb] = pltpu.async_copy(
        bufs[lb], out_hbm.at[pl.ds(base + (n_chunks - 1) * CH, CH)], osem[lb])
    for b in (0, 1):
      if sps[b] is not None:
        sps[b].wait()

  return pl.kernel(
      body,
      out_type=jax.ShapeDtypeStruct((n_rows, D), jnp.float32),
      mesh=mesh,
      scratch_types=[
          pltpu.VMEM((n_chunks, CH), jnp.int32),
          pltpu.VMEM((CH, D), jnp.float32),
          pltpu.VMEM((CH, D), jnp.float32),
          pltpu.SemaphoreType.DMA,
          pltpu.SemaphoreType.DMA,
          pltpu.SemaphoreType.DMA,
          pltpu.SemaphoreType.DMA,
      ],
  )


def _ffn_body(tid_ref, xs_ref, w1_ref, w3_ref, w2_ref, ws_ref, out_ref):
  xb = xs_ref[...]
  a = lax.dot_general(xb, w1_ref[0], (((1,), (1,)), ((), ())),
                      preferred_element_type=jnp.float32)
  b = lax.dot_general(xb, w3_ref[0], (((1,), (1,)), ((), ())),
                      preferred_element_type=jnp.float32)
  h = (a * jax.nn.sigmoid(a)) * b
  h = h * ws_ref[:, 0:1]
  out_ref[...] = lax.dot_general(h, w2_ref[0], (((1,), (1,)), ((), ())),
                                 preferred_element_type=jnp.float32)


def _ffn_call(teid, xs, w1, w3, w2, wsb):
  grid_spec = pltpu.PrefetchScalarGridSpec(
      num_scalar_prefetch=1,
      grid=(NT,),
      in_specs=[
          pl.BlockSpec((TM, D), lambda g, tid: (g, 0)),
          pl.BlockSpec((1, F, D), lambda g, tid: (tid[g], 0, 0)),
          pl.BlockSpec((1, F, D), lambda g, tid: (tid[g], 0, 0)),
          pl.BlockSpec((1, D, F), lambda g, tid: (tid[g], 0, 0)),
          pl.BlockSpec((TM, 128), lambda g, tid: (g, 0)),
      ],
      out_specs=pl.BlockSpec((TM, D), lambda g, tid: (g, 0)),
  )
  return pl.pallas_call(
      _ffn_body,
      grid_spec=grid_spec,
      out_shape=jax.ShapeDtypeStruct((G, D), jnp.float32),
      compiler_params=pltpu.CompilerParams(
          dimension_semantics=("arbitrary",)),
  )(teid, xs, w1, w3, w2, wsb)


def _add_body(a_ref, b_ref, out_ref):
  out_ref[...] = a_ref[...] + b_ref[...]


def _add_call(out01):
  nb = 16
  rb = T // nb
  return pl.pallas_call(
      _add_body,
      grid=(nb,),
      in_specs=[
          pl.BlockSpec((rb, D), lambda g: (g, 0)),
          pl.BlockSpec((rb, D), lambda g: (g + nb, 0)),
      ],
      out_specs=pl.BlockSpec((rb, D), lambda g: (g, 0)),
      out_shape=jax.ShapeDtypeStruct((T, D), jnp.float32),
  )(out01, out01)


def kernel(hidden_states, gate_w, e_score_correction_bias, w1, w2, w3):
  x = hidden_states.reshape(T, D)
  bias_b = jnp.broadcast_to(
      e_score_correction_bias.reshape(E, 1), (E, 128))
  pos, _wn, teid128, srci, wsf = _router_call(x, gate_w, bias_b)
  teid = teid128[0, :NT]
  src3d = srci.reshape(NW, G // (NW * CH), CH)
  xs = _make_sc_gather(G // (NW * CH))(x, src3d)
  wsb = jnp.broadcast_to(wsf, (G, 128))
  sorted_out = _ffn_call(teid, xs, w1, w3, w2, wsb)
  p3d = pos.reshape(NW, (K * T) // (NW * CH), CH)
  out01 = _make_sc_gather((K * T) // (NW * CH))(sorted_out, p3d)
  final = _add_call(out01)
  return final.reshape(1, T, D)
